# baseline (speedup 1.0000x reference)
import functools

import jax
import jax.numpy as jnp
from jax import lax
from jax.experimental import pallas as pl
from jax.experimental.pallas import tpu as pltpu
from jax.experimental.pallas import tpu_sc as plsc

_LANES = 16


def _gather_bias_t(table_flat, num_heads, side, win, a1, a2, rpt):
    info = plsc.get_sparse_core_info()
    nc = info.num_cores
    tbl_stride = side * side
    active = a1 // rpt
    row_words = num_heads * a2
    j0s = list(range(0, a2 - _LANES + 1, _LANES))
    if j0s[-1] + _LANES < a2:
        j0s.append(a2 - _LANES)
    mesh = plsc.VectorSubcoreMesh(core_axis_name="c", subcore_axis_name="s")

    @functools.partial(
        pl.kernel,
        out_type=jax.ShapeDtypeStruct((a1 * row_words,), jnp.float32),
        mesh=mesh,
        compiler_params=pltpu.CompilerParams(needs_layout_passes=False),
        scratch_types=[
            pltpu.VMEM((rpt * row_words,), jnp.float32),
            pltpu.VMEM((table_flat.shape[0],), jnp.float32),
        ],
    )
    def k(tbl_hbm, out_hbm, out_v, tbl_v):
        wid = lax.axis_index("s") * nc + lax.axis_index("c")

        @pl.when(wid < active)
        def _work():
            pltpu.sync_copy(tbl_hbm, tbl_v)
            r0 = wid * rpt
            lane = lax.iota(jnp.int32, _LANES)

            cvecs = []
            for j0 in j0s:
                m = j0 % win + lane
                d = jnp.full((_LANES,), j0 // win, jnp.int32)
                for _ in range(2):
                    w = m >= win
                    m = jnp.where(w, m - win, m)
                    d = d + w.astype(jnp.int32)
                cvecs.append(d * side + m)

            def row(r, carry):
                pi = r0 + r
                pidiv = pi // win
                pimod = pi - pidiv * win
                s0 = pidiv * side + pimod + (win - 1) * side + (win - 1)
                for jc, j0 in enumerate(j0s):
                    flat = s0 - cvecs[jc]
                    for h in range(num_heads):
                        vals = plsc.load_gather(tbl_v,
                                                [flat + h * tbl_stride])
                        out_v[pl.ds(r * row_words + h * a2 + j0,
                                    _LANES)] = vals
                return carry

            lax.fori_loop(0, rpt, row, 0)
            pltpu.sync_copy(out_v,
                            out_hbm.at[pl.ds(r0 * row_words,
                                             rpt * row_words)])

    return k(table_flat).reshape(a1, num_heads, a2)


def _gather_bias(table_flat, num_heads, side, win, a2, area_pad, per_tile):
    info = plsc.get_sparse_core_info()
    nc = info.num_cores
    chunks = per_tile // _LANES
    tbl_stride = side * side
    mesh = plsc.VectorSubcoreMesh(core_axis_name="c", subcore_axis_name="s")

    @functools.partial(
        pl.kernel,
        out_type=jax.ShapeDtypeStruct((num_heads * area_pad,), jnp.float32),
        mesh=mesh,
        compiler_params=pltpu.CompilerParams(needs_layout_passes=False),
        scratch_types=[
            pltpu.VMEM((num_heads * per_tile,), jnp.float32),
            pltpu.VMEM((table_flat.shape[0],), jnp.float32),
        ],
    )
    def k(tbl_hbm, out_hbm, out_v, tbl_v):
        wid = lax.axis_index("s") * nc + lax.axis_index("c")
        base = wid * per_tile
        pltpu.sync_copy(tbl_hbm, tbl_v)
        lane = lax.iota(jnp.int32, _LANES)

        pi0 = base // a2
        pj0 = base - pi0 * a2
        pidiv0 = pi0 // win
        pimod0 = pi0 - pidiv0 * win
        pjdiv0 = pj0 // win
        pjmod0 = pj0 - pjdiv0 * win

        def norm(mod, div, step=1):
            w = mod >= win
            return jnp.where(w, mod - win, mod), div + w.astype(jnp.int32)

        pjmod = pjmod0 + lane
        pjdiv = jnp.full((_LANES,), pjdiv0, jnp.int32)
        pjmod, pjdiv = norm(pjmod, pjdiv)
        pjmod, pjdiv = norm(pjmod, pjdiv)
        rowwrap = pjdiv >= (a2 // win)
        pjdiv = jnp.where(rowwrap, pjdiv - a2 // win, pjdiv)
        pimod = jnp.full((_LANES,), pimod0, jnp.int32) + rowwrap.astype(
            jnp.int32)
        pidiv = jnp.full((_LANES,), pidiv0, jnp.int32)
        pimod, pidiv = norm(pimod, pidiv)

        dmod = _LANES % win
        ddiv = _LANES // win

        def chunk(c, carry):
            pidiv, pimod, pjdiv, pjmod = carry
            i0 = pidiv - pjdiv + (win - 1)
            i1 = pimod - pjmod + (win - 1)
            flat = jnp.minimum(jnp.maximum(i0 * side + i1, 0),
                               tbl_stride - 1)
            for h in range(num_heads):
                vals = plsc.load_gather(tbl_v, [flat + h * tbl_stride])
                out_v[pl.ds(h * per_tile + c * _LANES, _LANES)] = vals
            pjmod = pjmod + dmod
            pjdiv = pjdiv + ddiv
            pjmod, pjdiv = norm(pjmod, pjdiv)
            rw = pjdiv >= (a2 // win)
            pjdiv = jnp.where(rw, pjdiv - a2 // win, pjdiv)
            pimod = pimod + rw.astype(jnp.int32)
            pimod, pidiv = norm(pimod, pidiv)
            return (pidiv, pimod, pjdiv, pjmod)

        lax.fori_loop(0, chunks, chunk, (pidiv, pimod, pjdiv, pjmod))
        for h in range(num_heads):
            pltpu.sync_copy(out_v.at[pl.ds(h * per_tile, per_tile)],
                            out_hbm.at[pl.ds(h * area_pad + base, per_tile)])

    return k(table_flat).reshape(num_heads, area_pad)


def _add_body_t(a_ref, b_ref, o_ref):
    b = jnp.transpose(b_ref[...], (1, 0, 2))
    o_ref[...] = a_ref[...] + b[..., None]


_NBUF = 4


def _make_stream_add(nw, nh, a1, a2, wpc):
    n_chunks = nw // wpc

    def body(attn_hbm, bias_ref, out_hbm, abuf, obuf, in_sems, out_sems):
        def start_in(c, b):
            pltpu.make_async_copy(attn_hbm.at[pl.ds(c * wpc, wpc)],
                                  abuf.at[b], in_sems.at[b]).start()

        for b in range(_NBUF):
            start_in(b, b)

        def outer(g, carry):
            for b in range(_NBUF):
                c = g * _NBUF + b
                pltpu.make_async_copy(attn_hbm.at[pl.ds(c * wpc, wpc)],
                                      abuf.at[b], in_sems.at[b]).wait()

                @pl.when(g > 0)
                def _wait_prev_out():
                    pltpu.make_async_copy(
                        obuf.at[b], out_hbm.at[pl.ds((c - _NBUF) * wpc, wpc)],
                        out_sems.at[b]).wait()

                obuf[b] = abuf[b] + bias_ref[...][None]

                @pl.when(c + _NBUF < n_chunks)
                def _prefetch():
                    start_in(c + _NBUF, b)

                pltpu.make_async_copy(obuf.at[b],
                                      out_hbm.at[pl.ds(c * wpc, wpc)],
                                      out_sems.at[b]).start()
            return carry

        lax.fori_loop(0, n_chunks // _NBUF, outer, 0)
        for b in range(_NBUF):
            c = n_chunks - _NBUF + b
            pltpu.make_async_copy(obuf.at[b],
                                  out_hbm.at[pl.ds(c * wpc, wpc)],
                                  out_sems.at[b]).wait()

    return pl.pallas_call(
        body,
        in_specs=[
            pl.BlockSpec(memory_space=pltpu.HBM),
            pl.BlockSpec(memory_space=pltpu.VMEM),
        ],
        out_specs=pl.BlockSpec(memory_space=pltpu.HBM),
        out_shape=jax.ShapeDtypeStruct((nw, nh, a1, a2), jnp.float32),
        scratch_shapes=[
            pltpu.VMEM((_NBUF, wpc, nh, a1, a2), jnp.float32),
            pltpu.VMEM((_NBUF, wpc, nh, a1, a2), jnp.float32),
            pltpu.SemaphoreType.DMA((_NBUF,)),
            pltpu.SemaphoreType.DMA((_NBUF,)),
        ],
    )


def kernel(attn, rel_pos_table, rel_pos_ind):
    nw, nh, a1, a2 = attn.shape
    area = a1 * a2
    side = rel_pos_table.shape[2]

    n_tiles = 32
    per_tile = -(-area // (n_tiles * _LANES)) * _LANES
    area_pad = n_tiles * per_tile
    win = a1
    while win * win > a1:
        win -= 1

    table_flat = rel_pos_table.reshape(-1)
    rpt = -(-a1 // n_tiles)
    if a1 % rpt == 0:
        bias_t = _gather_bias_t(table_flat, nh, side, win, a1, a2, rpt)
    else:
        bias_pad = _gather_bias(table_flat, nh, side, win, a2, area_pad,
                                per_tile)
        bias_t = jnp.transpose(bias_pad[:, :area].reshape(nh, a1, a2),
                               (1, 0, 2))

    attn_t = jnp.transpose(attn, (1, 2, 3, 0))
    bi = 7
    out_t = pl.pallas_call(
        _add_body_t,
        grid=(a1 // bi,),
        in_specs=[
            pl.BlockSpec((nh, bi, a2, nw), lambda i: (0, i, 0, 0)),
            pl.BlockSpec((bi, nh, a2), lambda i: (i, 0, 0)),
        ],
        out_specs=pl.BlockSpec((nh, bi, a2, nw), lambda i: (0, i, 0, 0)),
        out_shape=jax.ShapeDtypeStruct((nh, a1, a2, nw), jnp.float32),
    )(attn_t, bias_t)
    return jnp.transpose(out_t, (3, 0, 1, 2))


# speedup vs baseline: 1.0450x; 1.0450x over previous
import functools

import jax
import jax.numpy as jnp
from jax import lax
from jax.experimental import pallas as pl
from jax.experimental.pallas import tpu as pltpu
from jax.experimental.pallas import tpu_sc as plsc

_LANES = 16


def _gather_bias(table_flat, num_heads, side, win, a2, area_pad, per_tile):
    info = plsc.get_sparse_core_info()
    nc = info.num_cores
    chunks = per_tile // _LANES
    tbl_stride = side * side
    mesh = plsc.VectorSubcoreMesh(core_axis_name="c", subcore_axis_name="s")

    @functools.partial(
        pl.kernel,
        out_type=jax.ShapeDtypeStruct((num_heads * area_pad,), jnp.float32),
        mesh=mesh,
        compiler_params=pltpu.CompilerParams(needs_layout_passes=False),
        scratch_types=[
            pltpu.VMEM((num_heads * per_tile,), jnp.float32),
            pltpu.VMEM((table_flat.shape[0],), jnp.float32),
        ],
    )
    def k(tbl_hbm, out_hbm, out_v, tbl_v):
        wid = lax.axis_index("s") * nc + lax.axis_index("c")
        base = wid * per_tile
        pltpu.sync_copy(tbl_hbm, tbl_v)
        lane = lax.iota(jnp.int32, _LANES)

        pi0 = base // a2
        pj0 = base - pi0 * a2
        pidiv0 = pi0 // win
        pimod0 = pi0 - pidiv0 * win
        pjdiv0 = pj0 // win
        pjmod0 = pj0 - pjdiv0 * win

        def norm(mod, div, step=1):
            w = mod >= win
            return jnp.where(w, mod - win, mod), div + w.astype(jnp.int32)

        pjmod = pjmod0 + lane
        pjdiv = jnp.full((_LANES,), pjdiv0, jnp.int32)
        pjmod, pjdiv = norm(pjmod, pjdiv)
        pjmod, pjdiv = norm(pjmod, pjdiv)
        rowwrap = pjdiv >= (a2 // win)
        pjdiv = jnp.where(rowwrap, pjdiv - a2 // win, pjdiv)
        pimod = jnp.full((_LANES,), pimod0, jnp.int32) + rowwrap.astype(
            jnp.int32)
        pidiv = jnp.full((_LANES,), pidiv0, jnp.int32)
        pimod, pidiv = norm(pimod, pidiv)

        dmod = _LANES % win
        ddiv = _LANES // win

        def chunk(c, carry):
            pidiv, pimod, pjdiv, pjmod = carry
            i0 = pidiv - pjdiv + (win - 1)
            i1 = pimod - pjmod + (win - 1)
            flat = jnp.minimum(jnp.maximum(i0 * side + i1, 0),
                               tbl_stride - 1)
            for h in range(num_heads):
                vals = plsc.load_gather(tbl_v, [flat + h * tbl_stride])
                out_v[pl.ds(h * per_tile + c * _LANES, _LANES)] = vals
            pjmod = pjmod + dmod
            pjdiv = pjdiv + ddiv
            pjmod, pjdiv = norm(pjmod, pjdiv)
            rw = pjdiv >= (a2 // win)
            pjdiv = jnp.where(rw, pjdiv - a2 // win, pjdiv)
            pimod = pimod + rw.astype(jnp.int32)
            pimod, pidiv = norm(pimod, pidiv)
            return (pidiv, pimod, pjdiv, pjmod)

        lax.fori_loop(0, chunks, chunk, (pidiv, pimod, pjdiv, pjmod))
        for h in range(num_heads):
            pltpu.sync_copy(out_v.at[pl.ds(h * per_tile, per_tile)],
                            out_hbm.at[pl.ds(h * area_pad + base, per_tile)])

    return k(table_flat).reshape(num_heads, area_pad)


def _add_body_t(a_ref, b_ref, o_ref):
    b = jnp.transpose(b_ref[...], (1, 0, 2))
    o_ref[...] = a_ref[...] + b[..., None]


_NBUF = 4


def _make_stream_add(nw, nh, a1, a2, wpc):
    n_chunks = nw // wpc

    def body(attn_hbm, bias_ref, out_hbm, abuf, obuf, in_sems, out_sems):
        def start_in(c, b):
            pltpu.make_async_copy(attn_hbm.at[pl.ds(c * wpc, wpc)],
                                  abuf.at[b], in_sems.at[b]).start()

        for b in range(_NBUF):
            start_in(b, b)

        def outer(g, carry):
            for b in range(_NBUF):
                c = g * _NBUF + b
                pltpu.make_async_copy(attn_hbm.at[pl.ds(c * wpc, wpc)],
                                      abuf.at[b], in_sems.at[b]).wait()

                @pl.when(g > 0)
                def _wait_prev_out():
                    pltpu.make_async_copy(
                        obuf.at[b], out_hbm.at[pl.ds((c - _NBUF) * wpc, wpc)],
                        out_sems.at[b]).wait()

                obuf[b] = abuf[b] + bias_ref[...][None]

                @pl.when(c + _NBUF < n_chunks)
                def _prefetch():
                    start_in(c + _NBUF, b)

                pltpu.make_async_copy(obuf.at[b],
                                      out_hbm.at[pl.ds(c * wpc, wpc)],
                                      out_sems.at[b]).start()
            return carry

        lax.fori_loop(0, n_chunks // _NBUF, outer, 0)
        for b in range(_NBUF):
            c = n_chunks - _NBUF + b
            pltpu.make_async_copy(obuf.at[b],
                                  out_hbm.at[pl.ds(c * wpc, wpc)],
                                  out_sems.at[b]).wait()

    return pl.pallas_call(
        body,
        in_specs=[
            pl.BlockSpec(memory_space=pltpu.HBM),
            pl.BlockSpec(memory_space=pltpu.VMEM),
        ],
        out_specs=pl.BlockSpec(memory_space=pltpu.HBM),
        out_shape=jax.ShapeDtypeStruct((nw, nh, a1, a2), jnp.float32),
        scratch_shapes=[
            pltpu.VMEM((_NBUF, wpc, nh, a1, a2), jnp.float32),
            pltpu.VMEM((_NBUF, wpc, nh, a1, a2), jnp.float32),
            pltpu.SemaphoreType.DMA((_NBUF,)),
            pltpu.SemaphoreType.DMA((_NBUF,)),
        ],
    )


def kernel(attn, rel_pos_table, rel_pos_ind):
    nw, nh, a1, a2 = attn.shape
    area = a1 * a2
    side = rel_pos_table.shape[2]

    n_tiles = 32
    per_tile = -(-area // (n_tiles * _LANES)) * _LANES
    area_pad = n_tiles * per_tile
    win = a1
    while win * win > a1:
        win -= 1

    table_flat = rel_pos_table.reshape(-1)
    bias_pad = _gather_bias(table_flat, nh, side, win, a2, area_pad, per_tile)
    bias_t = jnp.transpose(bias_pad[:, :area].reshape(nh, a1, a2),
                           (1, 0, 2))

    attn_t = jnp.transpose(attn, (1, 2, 3, 0))
    bi = 7
    out_t = pl.pallas_call(
        _add_body_t,
        grid=(a1 // bi,),
        in_specs=[
            pl.BlockSpec((nh, bi, a2, nw), lambda i: (0, i, 0, 0)),
            pl.BlockSpec((bi, nh, a2), lambda i: (i, 0, 0)),
        ],
        out_specs=pl.BlockSpec((nh, bi, a2, nw), lambda i: (0, i, 0, 0)),
        out_shape=jax.ShapeDtypeStruct((nh, a1, a2, nw), jnp.float32),
    )(attn_t, bias_t)
    return jnp.transpose(out_t, (3, 0, 1, 2))
